# baseline (speedup 1.0000x reference)
import functools

import jax
import jax.numpy as jnp
from jax import lax
from jax.experimental import pallas as pl
from jax.experimental.pallas import tpu as pltpu, tpu_sc as plsc

B, T_PH, T_W = 16, 2048, 1024
EPS = 1e-05
L = 16
H = T_PH // 2
W_PAD = 1152
MAGIC = 12582912.0
MAGIC_BITS = 0x4B400000


def _body(ph_hbm, idx_hbm, wdp_hbm, out_hbm,
          ph_v, idx_v, wdp_v, seg_v, out_v, sem_ph, sem_ix, sem_wd):
    row = lax.axis_index("s")
    half = lax.axis_index("c")
    base = half * H

    cp_ph = pltpu.async_copy(ph_hbm.at[row], ph_v, sem_ph)
    cp_ix = pltpu.async_copy(idx_hbm.at[row], idx_v, sem_ix)
    cp_wd = pltpu.async_copy(wdp_hbm.at[row], wdp_v, sem_wd)

    zeros = jnp.zeros((L,), jnp.float32)

    @plsc.parallel_loop(0, W_PAD // L, unroll=8)
    def _(i):
        seg_v[pl.ds(i * L, L)] = zeros

    cp_ix.wait()
    cp_ph.wait()

    @plsc.parallel_loop(0, T_PH // L, unroll=8)
    def _(i):
        idx = idx_v[pl.ds(i * L, L)]
        vals = ph_v[pl.ds(i * L, L)]
        plsc.addupdate_scatter(seg_v, [idx], vals)

    cp_wd.wait()

    @plsc.parallel_loop(0, H // L, unroll=8)
    def _(i):
        off = base + i * L
        idx = idx_v[pl.ds(off, L)]
        vals = ph_v[pl.ds(off, L)]
        s = plsc.load_gather(seg_v, [idx])
        w = plsc.load_gather(wdp_v, [idx])
        x = vals * (w / jnp.maximum(s, EPS))
        y = x + MAGIC
        out_v[pl.ds(i * L, L)] = plsc.bitcast(y, jnp.int32) - MAGIC_BITS

    pltpu.sync_copy(out_v, out_hbm.at[row, pl.ds(base, H)])


@jax.jit
def _regulate(ph_dur, ph2word_i32, word_dur):
    mesh = plsc.VectorSubcoreMesh(core_axis_name="c", subcore_axis_name="s")
    f = functools.partial(
        pl.kernel,
        out_type=jax.ShapeDtypeStruct((B, T_PH), jnp.int32),
        mesh=mesh,
        compiler_params=pltpu.CompilerParams(needs_layout_passes=False),
        scratch_types=[
            pltpu.VMEM((T_PH,), jnp.float32),
            pltpu.VMEM((T_PH,), jnp.int32),
            pltpu.VMEM((W_PAD,), jnp.float32),
            pltpu.VMEM((W_PAD,), jnp.float32),
            pltpu.VMEM((H,), jnp.int32),
            pltpu.SemaphoreType.DMA,
            pltpu.SemaphoreType.DMA,
            pltpu.SemaphoreType.DMA,
        ],
    )(_body)
    wd_padded = jnp.pad(word_dur, ((0, 0), (1, W_PAD - 1 - T_W)))
    return f(ph_dur, ph2word_i32, wd_padded)


def kernel(ph_dur, ph2word, word_dur):
    out = _regulate(ph_dur.astype(jnp.float32), ph2word.astype(jnp.int32),
                    word_dur.astype(jnp.float32))
    return out.astype(jnp.int64)


# speedup vs baseline: 1.0002x; 1.0002x over previous
import functools

import jax
import jax.numpy as jnp
from jax import lax
from jax.experimental import pallas as pl
from jax.experimental.pallas import tpu as pltpu, tpu_sc as plsc

B, T_PH, T_W = 16, 2048, 1024
EPS = 1e-05
L = 16
H = T_PH // 2
W_PAD = 1152
MAGIC = 12582912.0
MAGIC_BITS = 0x4B400000


def _body(pk_hbm, wdp_hbm, out_hbm,
          pk_v, wdp_v, seg_v, out_v, sem_pk, sem_wd):
    row = lax.axis_index("s")
    half = lax.axis_index("c")
    base = half * H

    cp_pk = pltpu.async_copy(pk_hbm.at[row], pk_v, sem_pk)
    cp_wd = pltpu.async_copy(wdp_hbm.at[row], wdp_v, sem_wd)

    zeros = jnp.zeros((L,), jnp.float32)

    @plsc.parallel_loop(0, W_PAD // L, unroll=8)
    def _(i):
        seg_v[pl.ds(i * L, L)] = zeros

    cp_pk.wait()

    @plsc.parallel_loop(0, T_PH // L, unroll=8)
    def _(i):
        idx = plsc.bitcast(pk_v[pl.ds(T_PH + i * L, L)], jnp.int32)
        vals = pk_v[pl.ds(i * L, L)]
        plsc.addupdate_scatter(seg_v, [idx], vals)

    cp_wd.wait()

    @plsc.parallel_loop(0, H // L, unroll=8)
    def _(i):
        off = base + i * L
        idx = plsc.bitcast(pk_v[pl.ds(T_PH + off, L)], jnp.int32)
        vals = pk_v[pl.ds(off, L)]
        s = plsc.load_gather(seg_v, [idx])
        w = plsc.load_gather(wdp_v, [idx])
        x = vals * (w / jnp.maximum(s, EPS))
        y = x + MAGIC
        out_v[pl.ds(i * L, L)] = plsc.bitcast(y, jnp.int32) - MAGIC_BITS

    pltpu.sync_copy(out_v, out_hbm.at[row, pl.ds(base, H)])


@jax.jit
def _regulate(ph_dur, ph2word_i32, word_dur):
    mesh = plsc.VectorSubcoreMesh(core_axis_name="c", subcore_axis_name="s")
    f = functools.partial(
        pl.kernel,
        out_type=jax.ShapeDtypeStruct((B, T_PH), jnp.int32),
        mesh=mesh,
        compiler_params=pltpu.CompilerParams(needs_layout_passes=False),
        scratch_types=[
            pltpu.VMEM((2 * T_PH,), jnp.float32),
            pltpu.VMEM((W_PAD,), jnp.float32),
            pltpu.VMEM((W_PAD,), jnp.float32),
            pltpu.VMEM((H,), jnp.int32),
            pltpu.SemaphoreType.DMA,
            pltpu.SemaphoreType.DMA,
        ],
    )(_body)
    wd_padded = jnp.pad(word_dur, ((0, 0), (1, W_PAD - 1 - T_W)))
    packed = jnp.concatenate(
        [ph_dur, lax.bitcast_convert_type(ph2word_i32, jnp.float32)], axis=1)
    return f(packed, wd_padded)


def kernel(ph_dur, ph2word, word_dur):
    out = _regulate(ph_dur.astype(jnp.float32), ph2word.astype(jnp.int32),
                    word_dur.astype(jnp.float32))
    return out.astype(jnp.int64)


# speedup vs baseline: 1.0008x; 1.0006x over previous
import functools

import jax
import jax.numpy as jnp
from jax import lax
from jax.experimental import pallas as pl
from jax.experimental.pallas import tpu as pltpu, tpu_sc as plsc

B, T_PH, T_W = 16, 2048, 1024
EPS = 1e-05
L = 16
H = T_PH // 2
W_PAD = 1152
MAGIC = 12582912.0
MAGIC_BITS = 0x4B400000


def _body(ph_hbm, idx_hbm, wdp_hbm, out_hbm,
          ph_v, idx_v, wdp_v, seg_v, out_v, sem_ph, sem_ix, sem_wd):
    row = lax.axis_index("s")
    half = lax.axis_index("c")
    base = half * H

    cp_ph = pltpu.async_copy(ph_hbm.at[row], ph_v, sem_ph)
    cp_ix = pltpu.async_copy(idx_hbm.at[row], idx_v, sem_ix)
    cp_wd = pltpu.async_copy(wdp_hbm.at[row], wdp_v, sem_wd)

    zeros = jnp.zeros((L,), jnp.float32)

    @plsc.parallel_loop(0, W_PAD // L, unroll=8)
    def _(i):
        seg_v[pl.ds(i * L, L)] = zeros

    cp_ix.wait()
    cp_ph.wait()

    @plsc.parallel_loop(0, T_PH // L, unroll=8)
    def _(i):
        idx = idx_v[pl.ds(i * L, L)]
        vals = ph_v[pl.ds(i * L, L)]
        plsc.addupdate_scatter(seg_v, [idx], vals)

    cp_wd.wait()

    @plsc.parallel_loop(0, H // L, unroll=8)
    def _(i):
        off = base + i * L
        idx = idx_v[pl.ds(off, L)]
        vals = ph_v[pl.ds(off, L)]
        s = plsc.load_gather(seg_v, [idx])
        w = plsc.load_gather(wdp_v, [idx])
        x = vals * (w / jnp.maximum(s, EPS))
        y = x + MAGIC
        out_v[pl.ds(i * L, L)] = plsc.bitcast(y, jnp.int32) - MAGIC_BITS

    pltpu.sync_copy(out_v, out_hbm.at[row, pl.ds(base, H)])


@jax.jit
def _regulate(ph_dur, ph2word_i32, word_dur):
    mesh = plsc.VectorSubcoreMesh(core_axis_name="c", subcore_axis_name="s")
    f = functools.partial(
        pl.kernel,
        out_type=jax.ShapeDtypeStruct((B, T_PH), jnp.int32),
        mesh=mesh,
        compiler_params=pltpu.CompilerParams(needs_layout_passes=False),
        scratch_types=[
            pltpu.VMEM((T_PH,), jnp.float32),
            pltpu.VMEM((T_PH,), jnp.int32),
            pltpu.VMEM((W_PAD,), jnp.float32),
            pltpu.VMEM((W_PAD,), jnp.float32),
            pltpu.VMEM((H,), jnp.int32),
            pltpu.SemaphoreType.DMA,
            pltpu.SemaphoreType.DMA,
            pltpu.SemaphoreType.DMA,
        ],
    )(_body)
    wd_padded = jnp.pad(word_dur, ((0, 0), (1, W_PAD - 1 - T_W)))
    return f(ph_dur, ph2word_i32, wd_padded)


def kernel(ph_dur, ph2word, word_dur):
    out = _regulate(ph_dur.astype(jnp.float32), ph2word.astype(jnp.int32),
                    word_dur.astype(jnp.float32))
    return out.astype(jnp.int64)
